# Initial kernel scaffold; baseline (speedup 1.0000x reference)
import jax
import jax.numpy as jnp
from jax.experimental import pallas as pl


def kernel(tensor):
    raise NotImplementedError("write your pallas kernel here")


# speedup vs baseline: 1.3725x; 1.3725x over previous
import jax
import jax.numpy as jnp
from jax.experimental import pallas as pl


def _loss_kernel(x_ref, o_ref):
    x = x_ref[0]
    m = x[0]
    H, W = m.shape
    row = jax.lax.broadcasted_iota(jnp.int32, (H, W), 0)
    col = jax.lax.broadcasted_iota(jnp.int32, (H, W), 1)
    lin = row * W + col
    mv = jnp.max(m)
    idx = jnp.min(jnp.where(m == mv, lin, jnp.int32(H * W)))
    x0 = (idx // W).astype(jnp.float32)
    y0 = (idx % W).astype(jnp.float32)

    jj = row.astype(jnp.float32)
    kk = col.astype(jnp.float32)
    wgt = (x0 - jj) ** 2 + (y0 - kk) ** 2

    prod = x * wgt[:, :, None]
    o_ref[:] = jnp.sum(jnp.sum(prod, axis=0), axis=0, keepdims=True)


def kernel(tensor):
    B, C, H, W = tensor.shape
    out = pl.pallas_call(
        _loss_kernel,
        out_shape=jax.ShapeDtypeStruct((1, W), jnp.float32),
        grid=(1,),
        in_specs=[pl.BlockSpec((1, C, H, W), lambda i: (0, 0, 0, 0))],
        out_specs=pl.BlockSpec((1, W), lambda i: (0, 0)),
    )(tensor)
    return out[0]
